# baseline (speedup 1.0000x reference)
import functools

import jax
import jax.numpy as jnp
from jax import lax
from jax.experimental import pallas as pl
from jax.experimental.pallas import tpu as pltpu
from jax.experimental.pallas import tpu_sc as plsc

NC = 2
NS = 16
NW = NC * NS
L = 16
CH = 80


def _sc_mesh():
    return plsc.VectorSubcoreMesh(core_axis_name="c", subcore_axis_name="s")


def _deg_body(n, e, ei_hbm, deg_hbm, dall, dbuf, ones_v, zbuf, acc):
    ept = e // NW
    nch = ept // CH
    base_rows = (n // NS) // 8 * 8
    rem_rows = n - base_rows * NS
    cid = lax.axis_index("c")
    sid = lax.axis_index("s")
    wid = cid * NS + sid

    pltpu.sync_copy(ei_hbm.at[pl.ds(e + wid * ept, ept)], dall)

    zeros16 = jnp.zeros((L,), jnp.float32)
    ones16 = jnp.full((L,), 1.0, jnp.float32)
    for i in range(base_rows // L):
        zbuf[pl.ds(i * L, L)] = zeros16
    for i in range(CH // L):
        ones_v[pl.ds(i * L, L)] = ones16

    pltpu.sync_copy(zbuf, acc.at[pl.ds(sid * base_rows, base_rows)])

    @pl.when(sid == NS - 1)
    def _():
        pltpu.sync_copy(zbuf.at[pl.ds(0, rem_rows)],
                        acc.at[pl.ds(NS * base_rows, rem_rows)])

    plsc.subcore_barrier()

    def chunk(c, carry):
        for j in range(CH // L):
            dbuf[pl.ds(j * L, L)] = dall[pl.ds(c * CH + j * L, L)]
        pltpu.sync_copy(ones_v, acc.at[dbuf], add=True)
        return carry

    lax.fori_loop(0, nch, chunk, 0)
    plsc.subcore_barrier()

    pltpu.sync_copy(acc.at[pl.ds(sid * base_rows, base_rows)], zbuf)
    pltpu.sync_copy(zbuf,
                    deg_hbm.at[pl.ds(cid * n + sid * base_rows, base_rows)])

    @pl.when(sid == NS - 1)
    def _():
        pltpu.sync_copy(acc.at[pl.ds(NS * base_rows, rem_rows)],
                        zbuf.at[pl.ds(0, rem_rows)])
        pltpu.sync_copy(zbuf.at[pl.ds(0, rem_rows)],
                        deg_hbm.at[pl.ds(cid * n + NS * base_rows, rem_rows)])


NB = 3


def _edge_body(n, e, h, s_hbm, ei_hbm, out_hbm,
               sall, dbuf, rows, zrow, acc, *sems):
    ept = e // NW
    nch = ept // CH
    base_rows = (n // NS) // 8 * 8
    rem_rows = n - base_rows * NS
    cid = lax.axis_index("c")
    sid = lax.axis_index("s")
    wid = cid * NS + sid

    gsem = sems[:NB]
    dsem = sems[NB:2 * NB]
    ssem = sems[2 * NB:]

    pltpu.sync_copy(ei_hbm.at[pl.ds(wid * ept, ept)], sall)

    def issue(c, b):
        pltpu.async_copy(s_hbm.at[sall.at[pl.ds(c * CH, CH)]],
                         rows.at[b], gsem[b])
        pltpu.async_copy(ei_hbm.at[pl.ds(e + wid * ept + c * CH, CH)],
                         dbuf.at[b], dsem[b])

    def wait_gather(b):
        pltpu.make_async_copy(s_hbm.at[pl.ds(0, CH)],
                              rows.at[b], gsem[b]).wait()
        pltpu.make_async_copy(ei_hbm.at[pl.ds(0, CH)],
                              dbuf.at[b], dsem[b]).wait()

    def wait_scatter(b):
        pltpu.make_async_copy(s_hbm.at[pl.ds(0, CH)],
                              rows.at[b], ssem[b]).wait()

    for b in range(NB - 1):
        issue(b, b)

    zeros16 = jnp.zeros((L,), jnp.float32)
    for i in range(zrow.shape[0]):
        for j in range(h // L):
            zrow[i, pl.ds(j * L, L)] = zeros16

    zr = zrow.shape[0]

    def zchunk(k, carry):
        pltpu.sync_copy(zrow, acc.at[pl.ds(sid * base_rows + k * zr, zr)])
        return carry

    lax.fori_loop(0, base_rows // zr, zchunk, 0)

    @pl.when(sid == NS - 1)
    def _():
        for k in range(rem_rows // zr):
            pltpu.sync_copy(zrow, acc.at[pl.ds(NS * base_rows + k * zr, zr)])

    plsc.subcore_barrier()

    def visit(c, b, guard):
        wait_gather(b)
        pltpu.async_copy(rows.at[b], acc.at[dbuf.at[b]], ssem[b], add=True)
        bp = (b - 1) % NB
        cn = c + NB - 1 if not guard else None
        if guard:
            @pl.when(c + NB - 1 < nch)
            def _():
                @pl.when(c > 0)
                def _():
                    wait_scatter(bp)
                issue(c + NB - 1, bp)
        else:
            if cn < nch:
                if c > 0:
                    wait_scatter(bp)
                issue(cn, bp)

    def group(g, carry):
        for k in range(NB):
            visit(g * NB + k, k, True)
        return carry

    lax.fori_loop(0, nch // NB, group, 0)
    for c in range(nch - nch % NB, nch):
        visit(c, c % NB, False)
    for c in range(max(nch - NB, 0), nch):
        wait_scatter(c % NB)

    plsc.subcore_barrier()

    def wchunk(k, carry):
        pltpu.sync_copy(acc.at[pl.ds(sid * base_rows + k * zr, zr)], zrow)
        pltpu.sync_copy(zrow,
                        out_hbm.at[cid, pl.ds(sid * base_rows + k * zr, zr)])
        return carry

    lax.fori_loop(0, base_rows // zr, wchunk, 0)

    @pl.when(sid == NS - 1)
    def _():
        for k in range(rem_rows // zr):
            pltpu.sync_copy(acc.at[pl.ds(NS * base_rows + k * zr, zr)], zrow)
            pltpu.sync_copy(
                zrow, out_hbm.at[cid, pl.ds(NS * base_rows + k * zr, zr)])


def _sc_deg(ei, n):
    e = ei.shape[0] // 2
    base_rows = (n // NS) // 8 * 8
    k = pl.kernel(
        functools.partial(_deg_body, n, e),
        out_type=jax.ShapeDtypeStruct((NC * n,), jnp.float32),
        mesh=_sc_mesh(),
        scratch_types=[
            pltpu.VMEM((e // NW,), jnp.int32),
            pltpu.VMEM((CH,), jnp.int32),
            pltpu.VMEM((CH,), jnp.float32),
            pltpu.VMEM((base_rows,), jnp.float32),
            pltpu.VMEM_SHARED((n,), jnp.float32),
        ],
    )
    return k(ei)


def _sc_edge(s, ei):
    n, h = s.shape
    e = ei.shape[0] // 2
    k = pl.kernel(
        functools.partial(_edge_body, n, e, h),
        out_type=jax.ShapeDtypeStruct((NC, n, h), jnp.float32),
        mesh=_sc_mesh(),
        scratch_types=[
            pltpu.VMEM((e // NW,), jnp.int32),
            pltpu.VMEM((NB, CH), jnp.int32),
            pltpu.VMEM((NB, CH, h), jnp.float32),
            pltpu.VMEM((16, h), jnp.float32),
            pltpu.VMEM_SHARED((n, h), jnp.float32),
        ] + [pltpu.SemaphoreType.DMA] * (3 * NB),
    )
    return k(s, ei)


def _k1a_body(x_ref, w1_ref, u_ref):
    u_ref[...] = jnp.dot(x_ref[...], w1_ref[...],
                         preferred_element_type=jnp.float32)


def _k1b_body(dp_ref, u_ref, inv_ref, s1_ref):
    dp = dp_ref[...]
    deg = dp[0] + dp[1]
    inv = jnp.where(deg > 0, lax.rsqrt(jnp.maximum(deg, 1.0)), 0.0)
    inv_ref[...] = inv
    s1_ref[...] = u_ref[...] * inv


def _k3_body(p_ref, inv_ref, b1_ref, w2_ref, s2_ref):
    p = p_ref[...]
    inv = inv_ref[...]
    agg = (p[0] + p[1]) * inv + b1_ref[...]
    hid = jnp.where(agg >= 0, agg, 0.01 * agg)
    s2 = jnp.dot(hid, w2_ref[...], preferred_element_type=jnp.float32)
    s2_ref[...] = s2 * inv


def _k5_body(q_ref, inv_ref, b2_ref, wl_ref, bl_ref, o_ref):
    q = q_ref[...]
    agg = (q[0] + q[1]) * inv_ref[...] + b2_ref[...]
    hid = jnp.where(agg >= 0, agg, 0.01 * agg)
    o_ref[...] = jnp.dot(hid, wl_ref[...],
                         preferred_element_type=jnp.float32) + bl_ref[...]


def kernel(x, edge_index, W1, b1, W2, b2, W_lin, b_lin):
    n, d = x.shape
    h = W1.shape[1]
    lh = W2.shape[1]
    c = W_lin.shape[1]
    ei = edge_index.reshape(2 * edge_index.shape[1])

    bm = 2000
    grid = (n // bm,)

    u = pl.pallas_call(
        _k1a_body,
        grid=grid,
        in_specs=[
            pl.BlockSpec((bm, d), lambda m: (m, 0)),
            pl.BlockSpec((d, h), lambda m: (0, 0)),
        ],
        out_specs=pl.BlockSpec((bm, h), lambda m: (m, 0)),
        out_shape=jax.ShapeDtypeStruct((n, h), jnp.float32),
    )(x, W1)

    deg_p = _sc_deg(ei, n)
    dp = deg_p.reshape(NC, n, 1)

    inv, s1 = pl.pallas_call(
        _k1b_body,
        grid=grid,
        in_specs=[
            pl.BlockSpec((NC, bm, 1), lambda m: (0, m, 0)),
            pl.BlockSpec((bm, h), lambda m: (m, 0)),
        ],
        out_specs=[
            pl.BlockSpec((bm, 1), lambda m: (m, 0)),
            pl.BlockSpec((bm, h), lambda m: (m, 0)),
        ],
        out_shape=[
            jax.ShapeDtypeStruct((n, 1), jnp.float32),
            jax.ShapeDtypeStruct((n, h), jnp.float32),
        ],
    )(dp, u)

    p1 = _sc_edge(s1, ei)

    s2 = pl.pallas_call(
        _k3_body,
        grid=grid,
        in_specs=[
            pl.BlockSpec((NC, bm, h), lambda m: (0, m, 0)),
            pl.BlockSpec((bm, 1), lambda m: (m, 0)),
            pl.BlockSpec((1, h), lambda m: (0, 0)),
            pl.BlockSpec((h, lh), lambda m: (0, 0)),
        ],
        out_specs=pl.BlockSpec((bm, lh), lambda m: (m, 0)),
        out_shape=jax.ShapeDtypeStruct((n, lh), jnp.float32),
    )(p1, inv, b1.reshape(1, h), W2)

    p2 = _sc_edge(s2, ei)

    out = pl.pallas_call(
        _k5_body,
        grid=grid,
        in_specs=[
            pl.BlockSpec((NC, bm, lh), lambda m: (0, m, 0)),
            pl.BlockSpec((bm, 1), lambda m: (m, 0)),
            pl.BlockSpec((1, lh), lambda m: (0, 0)),
            pl.BlockSpec((lh, c), lambda m: (0, 0)),
            pl.BlockSpec((1, c), lambda m: (0, 0)),
        ],
        out_specs=pl.BlockSpec((bm, c), lambda m: (m, 0)),
        out_shape=jax.ShapeDtypeStruct((n, c), jnp.float32),
    )(p2, inv, b2.reshape(1, lh), W_lin, b_lin.reshape(1, c))

    return out


# speedup vs baseline: 1.0347x; 1.0347x over previous
import functools

import jax
import jax.numpy as jnp
from jax import lax
from jax.experimental import pallas as pl
from jax.experimental.pallas import tpu as pltpu
from jax.experimental.pallas import tpu_sc as plsc

NC = 2
NS = 16
NW = NC * NS
L = 16
CH = 80


def _sc_mesh():
    return plsc.VectorSubcoreMesh(core_axis_name="c", subcore_axis_name="s")


DNB = 5


def _deg_body(n, e, ei_hbm, deg_hbm, dall, dbuf, ones_v, zbuf, acc, *dsems):
    ept = e // NW
    nch = ept // CH
    base_rows = (n // NS) // 8 * 8
    rem_rows = n - base_rows * NS
    cid = lax.axis_index("c")
    sid = lax.axis_index("s")
    wid = cid * NS + sid

    pltpu.sync_copy(ei_hbm.at[pl.ds(e + wid * ept, ept)], dall)

    zeros16 = jnp.zeros((L,), jnp.float32)
    ones16 = jnp.full((L,), 1.0, jnp.float32)
    for i in range(base_rows // L):
        zbuf[pl.ds(i * L, L)] = zeros16
    for i in range(CH // L):
        ones_v[pl.ds(i * L, L)] = ones16

    pltpu.sync_copy(zbuf, acc.at[pl.ds(sid * base_rows, base_rows)])

    @pl.when(sid == NS - 1)
    def _():
        pltpu.sync_copy(zbuf.at[pl.ds(0, rem_rows)],
                        acc.at[pl.ds(NS * base_rows, rem_rows)])

    plsc.subcore_barrier()

    def group(g, carry):
        for b in range(DNB):
            c = g * DNB + b

            @pl.when(g > 0)
            def _():
                pltpu.make_async_copy(ei_hbm.at[pl.ds(0, CH)],
                                      dbuf.at[b], dsems[b]).wait()
            for j in range(CH // L):
                dbuf[b, pl.ds(j * L, L)] = dall[pl.ds(c * CH + j * L, L)]
            pltpu.async_copy(ones_v, acc.at[dbuf.at[b]], dsems[b], add=True)
        return carry

    lax.fori_loop(0, nch // DNB, group, 0)
    for b in range(DNB):
        pltpu.make_async_copy(ei_hbm.at[pl.ds(0, CH)],
                              dbuf.at[b], dsems[b]).wait()
    plsc.subcore_barrier()

    pltpu.sync_copy(acc.at[pl.ds(sid * base_rows, base_rows)], zbuf)
    pltpu.sync_copy(zbuf,
                    deg_hbm.at[pl.ds(cid * n + sid * base_rows, base_rows)])

    @pl.when(sid == NS - 1)
    def _():
        pltpu.sync_copy(acc.at[pl.ds(NS * base_rows, rem_rows)],
                        zbuf.at[pl.ds(0, rem_rows)])
        pltpu.sync_copy(zbuf.at[pl.ds(0, rem_rows)],
                        deg_hbm.at[pl.ds(cid * n + NS * base_rows, rem_rows)])


NB = 3


def _edge_body(n, e, h, s_hbm, ei_hbm, out_hbm,
               sall, dbuf, rows, zrow, acc, *sems):
    ept = e // NW
    nch = ept // CH
    base_rows = (n // NS) // 8 * 8
    rem_rows = n - base_rows * NS
    cid = lax.axis_index("c")
    sid = lax.axis_index("s")
    wid = cid * NS + sid

    gsem = sems[:NB]
    dsem = sems[NB:2 * NB]
    ssem = sems[2 * NB:]

    pltpu.sync_copy(ei_hbm.at[pl.ds(wid * ept, ept)], sall)

    def issue(c, b):
        pltpu.async_copy(s_hbm.at[sall.at[pl.ds(c * CH, CH)]],
                         rows.at[b], gsem[b])
        pltpu.async_copy(ei_hbm.at[pl.ds(e + wid * ept + c * CH, CH)],
                         dbuf.at[b], dsem[b])

    def wait_gather(b):
        pltpu.make_async_copy(s_hbm.at[pl.ds(0, CH)],
                              rows.at[b], gsem[b]).wait()
        pltpu.make_async_copy(ei_hbm.at[pl.ds(0, CH)],
                              dbuf.at[b], dsem[b]).wait()

    def wait_scatter(b):
        pltpu.make_async_copy(s_hbm.at[pl.ds(0, CH)],
                              rows.at[b], ssem[b]).wait()

    for b in range(NB - 1):
        issue(b, b)

    zeros16 = jnp.zeros((L,), jnp.float32)
    for i in range(zrow.shape[0]):
        for j in range(h // L):
            zrow[i, pl.ds(j * L, L)] = zeros16

    zr = zrow.shape[0]

    def zchunk(k, carry):
        pltpu.sync_copy(zrow, acc.at[pl.ds(sid * base_rows + k * zr, zr)])
        return carry

    lax.fori_loop(0, base_rows // zr, zchunk, 0)

    @pl.when(sid == NS - 1)
    def _():
        for k in range(rem_rows // zr):
            pltpu.sync_copy(zrow, acc.at[pl.ds(NS * base_rows + k * zr, zr)])

    plsc.subcore_barrier()

    def visit(c, b, guard):
        wait_gather(b)
        pltpu.async_copy(rows.at[b], acc.at[dbuf.at[b]], ssem[b], add=True)
        bp = (b - 1) % NB
        cn = c + NB - 1 if not guard else None
        if guard:
            @pl.when(c + NB - 1 < nch)
            def _():
                @pl.when(c > 0)
                def _():
                    wait_scatter(bp)
                issue(c + NB - 1, bp)
        else:
            if cn < nch:
                if c > 0:
                    wait_scatter(bp)
                issue(cn, bp)

    def group(g, carry):
        for k in range(NB):
            visit(g * NB + k, k, True)
        return carry

    lax.fori_loop(0, nch // NB, group, 0)
    for c in range(nch - nch % NB, nch):
        visit(c, c % NB, False)
    for c in range(max(nch - NB, 0), nch):
        wait_scatter(c % NB)

    plsc.subcore_barrier()

    def wchunk(k, carry):
        pltpu.sync_copy(acc.at[pl.ds(sid * base_rows + k * zr, zr)], zrow)
        pltpu.sync_copy(zrow,
                        out_hbm.at[cid, pl.ds(sid * base_rows + k * zr, zr)])
        return carry

    lax.fori_loop(0, base_rows // zr, wchunk, 0)

    @pl.when(sid == NS - 1)
    def _():
        for k in range(rem_rows // zr):
            pltpu.sync_copy(acc.at[pl.ds(NS * base_rows + k * zr, zr)], zrow)
            pltpu.sync_copy(
                zrow, out_hbm.at[cid, pl.ds(NS * base_rows + k * zr, zr)])


def _sc_deg(ei, n):
    e = ei.shape[0] // 2
    base_rows = (n // NS) // 8 * 8
    k = pl.kernel(
        functools.partial(_deg_body, n, e),
        out_type=jax.ShapeDtypeStruct((NC * n,), jnp.float32),
        mesh=_sc_mesh(),
        scratch_types=[
            pltpu.VMEM((e // NW,), jnp.int32),
            pltpu.VMEM((DNB, CH), jnp.int32),
            pltpu.VMEM((CH,), jnp.float32),
            pltpu.VMEM((base_rows,), jnp.float32),
            pltpu.VMEM_SHARED((n,), jnp.float32),
        ] + [pltpu.SemaphoreType.DMA] * DNB,
    )
    return k(ei)


def _sc_edge(s, ei):
    n, h = s.shape
    e = ei.shape[0] // 2
    k = pl.kernel(
        functools.partial(_edge_body, n, e, h),
        out_type=jax.ShapeDtypeStruct((NC, n, h), jnp.float32),
        mesh=_sc_mesh(),
        scratch_types=[
            pltpu.VMEM((e // NW,), jnp.int32),
            pltpu.VMEM((NB, CH), jnp.int32),
            pltpu.VMEM((NB, CH, h), jnp.float32),
            pltpu.VMEM((16, h), jnp.float32),
            pltpu.VMEM_SHARED((n, h), jnp.float32),
        ] + [pltpu.SemaphoreType.DMA] * (3 * NB),
    )
    return k(s, ei)


def _k1a_body(x_ref, w1_ref, u_ref):
    u_ref[...] = jnp.dot(x_ref[...], w1_ref[...],
                         preferred_element_type=jnp.float32)


def _k1b_body(dp_ref, u_ref, inv_ref, s1_ref):
    dp = dp_ref[...]
    deg = dp[0] + dp[1]
    inv = jnp.where(deg > 0, lax.rsqrt(jnp.maximum(deg, 1.0)), 0.0)
    inv_ref[...] = inv
    s1_ref[...] = u_ref[...] * inv


def _k3_body(p_ref, inv_ref, b1_ref, w2_ref, s2_ref):
    p = p_ref[...]
    inv = inv_ref[...]
    agg = (p[0] + p[1]) * inv + b1_ref[...]
    hid = jnp.where(agg >= 0, agg, 0.01 * agg)
    s2 = jnp.dot(hid, w2_ref[...], preferred_element_type=jnp.float32)
    s2_ref[...] = s2 * inv


def _k5_body(q_ref, inv_ref, b2_ref, wl_ref, bl_ref, o_ref):
    q = q_ref[...]
    agg = (q[0] + q[1]) * inv_ref[...] + b2_ref[...]
    hid = jnp.where(agg >= 0, agg, 0.01 * agg)
    o_ref[...] = jnp.dot(hid, wl_ref[...],
                         preferred_element_type=jnp.float32) + bl_ref[...]


def kernel(x, edge_index, W1, b1, W2, b2, W_lin, b_lin):
    n, d = x.shape
    h = W1.shape[1]
    lh = W2.shape[1]
    c = W_lin.shape[1]
    ei = edge_index.reshape(2 * edge_index.shape[1])

    bm = 2000
    grid = (n // bm,)

    u = pl.pallas_call(
        _k1a_body,
        grid=grid,
        in_specs=[
            pl.BlockSpec((bm, d), lambda m: (m, 0)),
            pl.BlockSpec((d, h), lambda m: (0, 0)),
        ],
        out_specs=pl.BlockSpec((bm, h), lambda m: (m, 0)),
        out_shape=jax.ShapeDtypeStruct((n, h), jnp.float32),
    )(x, W1)

    deg_p = _sc_deg(ei, n)
    dp = deg_p.reshape(NC, n, 1)

    inv, s1 = pl.pallas_call(
        _k1b_body,
        grid=grid,
        in_specs=[
            pl.BlockSpec((NC, bm, 1), lambda m: (0, m, 0)),
            pl.BlockSpec((bm, h), lambda m: (m, 0)),
        ],
        out_specs=[
            pl.BlockSpec((bm, 1), lambda m: (m, 0)),
            pl.BlockSpec((bm, h), lambda m: (m, 0)),
        ],
        out_shape=[
            jax.ShapeDtypeStruct((n, 1), jnp.float32),
            jax.ShapeDtypeStruct((n, h), jnp.float32),
        ],
    )(dp, u)

    p1 = _sc_edge(s1, ei)

    s2 = pl.pallas_call(
        _k3_body,
        grid=grid,
        in_specs=[
            pl.BlockSpec((NC, bm, h), lambda m: (0, m, 0)),
            pl.BlockSpec((bm, 1), lambda m: (m, 0)),
            pl.BlockSpec((1, h), lambda m: (0, 0)),
            pl.BlockSpec((h, lh), lambda m: (0, 0)),
        ],
        out_specs=pl.BlockSpec((bm, lh), lambda m: (m, 0)),
        out_shape=jax.ShapeDtypeStruct((n, lh), jnp.float32),
    )(p1, inv, b1.reshape(1, h), W2)

    p2 = _sc_edge(s2, ei)

    out = pl.pallas_call(
        _k5_body,
        grid=grid,
        in_specs=[
            pl.BlockSpec((NC, bm, lh), lambda m: (0, m, 0)),
            pl.BlockSpec((bm, 1), lambda m: (m, 0)),
            pl.BlockSpec((1, lh), lambda m: (0, 0)),
            pl.BlockSpec((lh, c), lambda m: (0, 0)),
            pl.BlockSpec((1, c), lambda m: (0, 0)),
        ],
        out_specs=pl.BlockSpec((bm, c), lambda m: (m, 0)),
        out_shape=jax.ShapeDtypeStruct((n, c), jnp.float32),
    )(p2, inv, b2.reshape(1, lh), W_lin, b_lin.reshape(1, c))

    return out


# speedup vs baseline: 1.0526x; 1.0173x over previous
import functools

import jax
import jax.numpy as jnp
from jax import lax
from jax.experimental import pallas as pl
from jax.experimental.pallas import tpu as pltpu
from jax.experimental.pallas import tpu_sc as plsc

NC = 2
NS = 16
NW = NC * NS
L = 16
CH = 80


def _sc_mesh():
    return plsc.VectorSubcoreMesh(core_axis_name="c", subcore_axis_name="s")


DNB = 5


def _deg_body(n, e, ei_hbm, deg_hbm, dall, dbuf, ones_v, zbuf, acc, *dsems):
    ept = e // NW
    nch = ept // CH
    base_rows = (n // NS) // 8 * 8
    rem_rows = n - base_rows * NS
    cid = lax.axis_index("c")
    sid = lax.axis_index("s")
    wid = cid * NS + sid

    pltpu.sync_copy(ei_hbm.at[pl.ds(e + wid * ept, ept)], dall)

    zeros16 = jnp.zeros((L,), jnp.float32)
    ones16 = jnp.full((L,), 1.0, jnp.float32)
    for i in range(base_rows // L):
        zbuf[pl.ds(i * L, L)] = zeros16
    for i in range(CH // L):
        ones_v[pl.ds(i * L, L)] = ones16

    pltpu.sync_copy(zbuf, acc.at[pl.ds(sid * base_rows, base_rows)])

    @pl.when(sid == NS - 1)
    def _():
        pltpu.sync_copy(zbuf.at[pl.ds(0, rem_rows)],
                        acc.at[pl.ds(NS * base_rows, rem_rows)])

    plsc.subcore_barrier()

    def group(g, carry):
        for b in range(DNB):
            c = g * DNB + b

            @pl.when(g > 0)
            def _():
                pltpu.make_async_copy(ei_hbm.at[pl.ds(0, CH)],
                                      dbuf.at[b], dsems[b]).wait()
            for j in range(CH // L):
                dbuf[b, pl.ds(j * L, L)] = dall[pl.ds(c * CH + j * L, L)]
            pltpu.async_copy(ones_v, acc.at[dbuf.at[b]], dsems[b], add=True)
        return carry

    lax.fori_loop(0, nch // DNB, group, 0)
    for b in range(DNB):
        pltpu.make_async_copy(ei_hbm.at[pl.ds(0, CH)],
                              dbuf.at[b], dsems[b]).wait()
    plsc.subcore_barrier()

    pltpu.sync_copy(acc.at[pl.ds(sid * base_rows, base_rows)], zbuf)
    pltpu.sync_copy(zbuf,
                    deg_hbm.at[pl.ds(cid * n + sid * base_rows, base_rows)])

    @pl.when(sid == NS - 1)
    def _():
        pltpu.sync_copy(acc.at[pl.ds(NS * base_rows, rem_rows)],
                        zbuf.at[pl.ds(0, rem_rows)])
        pltpu.sync_copy(zbuf.at[pl.ds(0, rem_rows)],
                        deg_hbm.at[pl.ds(cid * n + NS * base_rows, rem_rows)])


NB = 5
ECH = 40


def _edge_body(n, e, h, s_hbm, ei_hbm, out_hbm,
               sall, dbuf, rows, zrow, acc, *sems):
    ept = e // NW
    nch = ept // ECH
    base_rows = (n // NS) // 8 * 8
    rem_rows = n - base_rows * NS
    cid = lax.axis_index("c")
    sid = lax.axis_index("s")
    wid = cid * NS + sid

    gsem = sems[:NB]
    dsem = sems[NB:2 * NB]
    ssem = sems[2 * NB:]

    pltpu.sync_copy(ei_hbm.at[pl.ds(wid * ept, ept)], sall)

    def issue(c, b):
        pltpu.async_copy(s_hbm.at[sall.at[pl.ds(c * ECH, ECH)]],
                         rows.at[b], gsem[b])
        pltpu.async_copy(ei_hbm.at[pl.ds(e + wid * ept + c * ECH, ECH)],
                         dbuf.at[b], dsem[b])

    def wait_gather(b):
        pltpu.make_async_copy(s_hbm.at[pl.ds(0, ECH)],
                              rows.at[b], gsem[b]).wait()
        pltpu.make_async_copy(ei_hbm.at[pl.ds(0, ECH)],
                              dbuf.at[b], dsem[b]).wait()

    def wait_scatter(b):
        pltpu.make_async_copy(s_hbm.at[pl.ds(0, ECH)],
                              rows.at[b], ssem[b]).wait()

    for b in range(NB - 1):
        issue(b, b)

    zeros16 = jnp.zeros((L,), jnp.float32)
    for i in range(zrow.shape[0]):
        for j in range(h // L):
            zrow[i, pl.ds(j * L, L)] = zeros16

    zr = zrow.shape[0]

    def zchunk(k, carry):
        pltpu.sync_copy(zrow, acc.at[pl.ds(sid * base_rows + k * zr, zr)])
        return carry

    lax.fori_loop(0, base_rows // zr, zchunk, 0)

    @pl.when(sid == NS - 1)
    def _():
        for k in range(rem_rows // zr):
            pltpu.sync_copy(zrow, acc.at[pl.ds(NS * base_rows + k * zr, zr)])

    plsc.subcore_barrier()

    def visit(c, b, guard):
        wait_gather(b)
        pltpu.async_copy(rows.at[b], acc.at[dbuf.at[b]], ssem[b], add=True)
        bp = (b - 1) % NB
        cn = c + NB - 1 if not guard else None
        if guard:
            @pl.when(c + NB - 1 < nch)
            def _():
                @pl.when(c > 0)
                def _():
                    wait_scatter(bp)
                issue(c + NB - 1, bp)
        else:
            if cn < nch:
                if c > 0:
                    wait_scatter(bp)
                issue(cn, bp)

    def group(g, carry):
        for k in range(NB):
            visit(g * NB + k, k, True)
        return carry

    lax.fori_loop(0, nch // NB, group, 0)
    for c in range(nch - nch % NB, nch):
        visit(c, c % NB, False)
    for c in range(max(nch - NB, 0), nch):
        wait_scatter(c % NB)

    plsc.subcore_barrier()

    def wchunk(k, carry):
        pltpu.sync_copy(acc.at[pl.ds(sid * base_rows + k * zr, zr)], zrow)
        pltpu.sync_copy(zrow,
                        out_hbm.at[cid, pl.ds(sid * base_rows + k * zr, zr)])
        return carry

    lax.fori_loop(0, base_rows // zr, wchunk, 0)

    @pl.when(sid == NS - 1)
    def _():
        for k in range(rem_rows // zr):
            pltpu.sync_copy(acc.at[pl.ds(NS * base_rows + k * zr, zr)], zrow)
            pltpu.sync_copy(
                zrow, out_hbm.at[cid, pl.ds(NS * base_rows + k * zr, zr)])


def _sc_deg(ei, n):
    e = ei.shape[0] // 2
    base_rows = (n // NS) // 8 * 8
    k = pl.kernel(
        functools.partial(_deg_body, n, e),
        out_type=jax.ShapeDtypeStruct((NC * n,), jnp.float32),
        mesh=_sc_mesh(),
        scratch_types=[
            pltpu.VMEM((e // NW,), jnp.int32),
            pltpu.VMEM((DNB, CH), jnp.int32),
            pltpu.VMEM((CH,), jnp.float32),
            pltpu.VMEM((base_rows,), jnp.float32),
            pltpu.VMEM_SHARED((n,), jnp.float32),
        ] + [pltpu.SemaphoreType.DMA] * DNB,
    )
    return k(ei)


def _sc_edge(s, ei):
    n, h = s.shape
    e = ei.shape[0] // 2
    k = pl.kernel(
        functools.partial(_edge_body, n, e, h),
        out_type=jax.ShapeDtypeStruct((NC, n, h), jnp.float32),
        mesh=_sc_mesh(),
        scratch_types=[
            pltpu.VMEM((e // NW,), jnp.int32),
            pltpu.VMEM((NB, ECH), jnp.int32),
            pltpu.VMEM((NB, ECH, h), jnp.float32),
            pltpu.VMEM((16, h), jnp.float32),
            pltpu.VMEM_SHARED((n, h), jnp.float32),
        ] + [pltpu.SemaphoreType.DMA] * (3 * NB),
    )
    return k(s, ei)


def _k1a_body(x_ref, w1_ref, u_ref):
    u_ref[...] = jnp.dot(x_ref[...], w1_ref[...],
                         preferred_element_type=jnp.float32)


def _k1b_body(dp_ref, u_ref, inv_ref, s1_ref):
    dp = dp_ref[...]
    deg = dp[0] + dp[1]
    inv = jnp.where(deg > 0, lax.rsqrt(jnp.maximum(deg, 1.0)), 0.0)
    inv_ref[...] = inv
    s1_ref[...] = u_ref[...] * inv


def _k3_body(p_ref, inv_ref, b1_ref, w2_ref, s2_ref):
    p = p_ref[...]
    inv = inv_ref[...]
    agg = (p[0] + p[1]) * inv + b1_ref[...]
    hid = jnp.where(agg >= 0, agg, 0.01 * agg)
    s2 = jnp.dot(hid, w2_ref[...], preferred_element_type=jnp.float32)
    s2_ref[...] = s2 * inv


def _k5_body(q_ref, inv_ref, b2_ref, wl_ref, bl_ref, o_ref):
    q = q_ref[...]
    agg = (q[0] + q[1]) * inv_ref[...] + b2_ref[...]
    hid = jnp.where(agg >= 0, agg, 0.01 * agg)
    o_ref[...] = jnp.dot(hid, wl_ref[...],
                         preferred_element_type=jnp.float32) + bl_ref[...]


def kernel(x, edge_index, W1, b1, W2, b2, W_lin, b_lin):
    n, d = x.shape
    h = W1.shape[1]
    lh = W2.shape[1]
    c = W_lin.shape[1]
    ei = edge_index.reshape(2 * edge_index.shape[1])

    bm = 2000
    grid = (n // bm,)

    u = pl.pallas_call(
        _k1a_body,
        grid=grid,
        in_specs=[
            pl.BlockSpec((bm, d), lambda m: (m, 0)),
            pl.BlockSpec((d, h), lambda m: (0, 0)),
        ],
        out_specs=pl.BlockSpec((bm, h), lambda m: (m, 0)),
        out_shape=jax.ShapeDtypeStruct((n, h), jnp.float32),
    )(x, W1)

    deg_p = _sc_deg(ei, n)
    dp = deg_p.reshape(NC, n, 1)

    inv, s1 = pl.pallas_call(
        _k1b_body,
        grid=grid,
        in_specs=[
            pl.BlockSpec((NC, bm, 1), lambda m: (0, m, 0)),
            pl.BlockSpec((bm, h), lambda m: (m, 0)),
        ],
        out_specs=[
            pl.BlockSpec((bm, 1), lambda m: (m, 0)),
            pl.BlockSpec((bm, h), lambda m: (m, 0)),
        ],
        out_shape=[
            jax.ShapeDtypeStruct((n, 1), jnp.float32),
            jax.ShapeDtypeStruct((n, h), jnp.float32),
        ],
    )(dp, u)

    p1 = _sc_edge(s1, ei)

    s2 = pl.pallas_call(
        _k3_body,
        grid=grid,
        in_specs=[
            pl.BlockSpec((NC, bm, h), lambda m: (0, m, 0)),
            pl.BlockSpec((bm, 1), lambda m: (m, 0)),
            pl.BlockSpec((1, h), lambda m: (0, 0)),
            pl.BlockSpec((h, lh), lambda m: (0, 0)),
        ],
        out_specs=pl.BlockSpec((bm, lh), lambda m: (m, 0)),
        out_shape=jax.ShapeDtypeStruct((n, lh), jnp.float32),
    )(p1, inv, b1.reshape(1, h), W2)

    p2 = _sc_edge(s2, ei)

    out = pl.pallas_call(
        _k5_body,
        grid=grid,
        in_specs=[
            pl.BlockSpec((NC, bm, lh), lambda m: (0, m, 0)),
            pl.BlockSpec((bm, 1), lambda m: (m, 0)),
            pl.BlockSpec((1, lh), lambda m: (0, 0)),
            pl.BlockSpec((lh, c), lambda m: (0, 0)),
            pl.BlockSpec((1, c), lambda m: (0, 0)),
        ],
        out_specs=pl.BlockSpec((bm, c), lambda m: (m, 0)),
        out_shape=jax.ShapeDtypeStruct((n, c), jnp.float32),
    )(p2, inv, b2.reshape(1, lh), W_lin, b_lin.reshape(1, c))

    return out
